# baseline (speedup 1.0000x reference)
import jax
import jax.numpy as jnp
from jax.experimental import pallas as pl

_BATCH_TILE = 8


def _chamfer_body(a_ref, b_ref, x1_ref, y1_ref, x2_ref, y2_ref, out_ref):
    vals = []
    for t in range(_BATCH_TILE):
        lhs = a_ref[t]
        rhs = b_ref[t]
        c2 = jax.lax.dot_general(
            lhs, rhs, (((0,), (0,)), ((), ())),
            preferred_element_type=jnp.float32,
        )
        x1 = x1_ref[t, 0, :]
        y1 = y1_ref[t, 0, :]
        x2 = x2_ref[t, 0, :]
        y2 = y2_ref[t, 0, :]
        a = x1 * x1 + y1 * y1
        b = x2 * x2 + y2 * y2
        rmin = a + jnp.min(b[None, :] - c2, axis=1)
        cmin = b + jnp.min(a[:, None] - c2, axis=0)
        d_fwd = jnp.mean(jnp.sqrt(jnp.maximum(rmin, 0.0)))
        d_bwd = jnp.mean(jnp.sqrt(jnp.maximum(cmin, 0.0)))
        vals.append((d_fwd + d_bwd) * 0.5)
    out_ref[...] = jnp.stack(vals).reshape(_BATCH_TILE, 1, 1)


def _hilo(v):
    h = jax.lax.optimization_barrier(v.astype(jnp.bfloat16))
    l = (v - h.astype(jnp.float32)).astype(jnp.bfloat16)
    return h, l


def kernel(point_set_1, point_set_2):
    b, p1, _ = point_set_1.shape
    p2 = point_set_2.shape[1]
    t = _BATCH_TILE
    x1 = point_set_1[:, :, 0]
    y1 = point_set_1[:, :, 1]
    x2 = point_set_2[:, :, 0]
    y2 = point_set_2[:, :, 1]
    hx1, lx1 = _hilo(x1 * 2.0)
    hy1, ly1 = _hilo(y1 * 2.0)
    hx2, lx2 = _hilo(x2)
    hy2, ly2 = _hilo(y2)
    z = jnp.zeros_like(hx1)
    lhs = jnp.stack([hx1, hx1, lx1, hy1, hy1, ly1, z, z], axis=1)
    rhs = jnp.stack([hx2, lx2, hx2, hy2, ly2, hy2, z, z], axis=1)
    out = pl.pallas_call(
        _chamfer_body,
        grid=(b // t,),
        in_specs=[
            pl.BlockSpec((t, 8, p1), lambda i: (i, 0, 0)),
            pl.BlockSpec((t, 8, p2), lambda i: (i, 0, 0)),
            pl.BlockSpec((t, 1, p1), lambda i: (i, 0, 0)),
            pl.BlockSpec((t, 1, p1), lambda i: (i, 0, 0)),
            pl.BlockSpec((t, 1, p2), lambda i: (i, 0, 0)),
            pl.BlockSpec((t, 1, p2), lambda i: (i, 0, 0)),
        ],
        out_specs=pl.BlockSpec((t, 1, 1), lambda i: (i, 0, 0)),
        out_shape=jax.ShapeDtypeStruct((b, 1, 1), jnp.float32),
    )(lhs, rhs,
      x1.reshape(b, 1, p1), y1.reshape(b, 1, p1),
      x2.reshape(b, 1, p2), y2.reshape(b, 1, p2))
    return out[:, 0, 0]


# speedup vs baseline: 1.7006x; 1.7006x over previous
import jax
import jax.numpy as jnp
from jax.experimental import pallas as pl

_BATCH_TILE = 16
_TINY = 1e-30


def _chamfer_body(x1_ref, y1_ref, x2_ref, y2_ref, out_ref):
    rmins = []
    bwd = []
    for t in range(_BATCH_TILE):
        x1 = x1_ref[t, 0, :]
        y1 = y1_ref[t, 0, :]
        x2 = x2_ref[t, 0, :]
        y2 = y2_ref[t, 0, :]
        dx = x1[:, None] - x2[None, :]
        dy = y1[:, None] - y2[None, :]
        d2 = dx * dx + dy * dy
        rmins.append(jnp.min(d2, axis=1, keepdims=True))
        cmin = jnp.min(d2, axis=0)
        bwd.append(jnp.mean(cmin * jax.lax.rsqrt(cmin + _TINY)))
    r = jnp.concatenate(rmins, axis=1)
    rt = r.T
    d_fwd = jnp.mean(rt * jax.lax.rsqrt(rt + _TINY), axis=1)
    d_bwd = jnp.stack(bwd)
    out_ref[...] = ((d_fwd + d_bwd) * 0.5).reshape(_BATCH_TILE, 1, 1)


def kernel(point_set_1, point_set_2):
    b, p1, _ = point_set_1.shape
    p2 = point_set_2.shape[1]
    t = _BATCH_TILE
    x1 = point_set_1[:, :, 0].reshape(b, 1, p1)
    y1 = point_set_1[:, :, 1].reshape(b, 1, p1)
    x2 = point_set_2[:, :, 0].reshape(b, 1, p2)
    y2 = point_set_2[:, :, 1].reshape(b, 1, p2)
    out = pl.pallas_call(
        _chamfer_body,
        grid=(b // t,),
        in_specs=[
            pl.BlockSpec((t, 1, p1), lambda i: (i, 0, 0)),
            pl.BlockSpec((t, 1, p1), lambda i: (i, 0, 0)),
            pl.BlockSpec((t, 1, p2), lambda i: (i, 0, 0)),
            pl.BlockSpec((t, 1, p2), lambda i: (i, 0, 0)),
        ],
        out_specs=pl.BlockSpec((t, 1, 1), lambda i: (i, 0, 0)),
        out_shape=jax.ShapeDtypeStruct((b, 1, 1), jnp.float32),
    )(x1, y1, x2, y2)
    return out[:, 0, 0]


# speedup vs baseline: 1.7356x; 1.0206x over previous
import jax
import jax.numpy as jnp
from jax.experimental import pallas as pl

_BATCH_TILE = 16
_TINY = 1e-30


def _chamfer_body(c_ref, out_ref):
    rmins = []
    bwd = []
    for t in range(_BATCH_TILE):
        x1 = c_ref[0, t, :]
        x2 = c_ref[1, t, :]
        y1 = c_ref[2, t, :]
        y2 = c_ref[3, t, :]
        dx = x1[:, None] - x2[None, :]
        dy = y1[:, None] - y2[None, :]
        d2 = dx * dx + dy * dy
        rmins.append(jnp.min(d2, axis=1, keepdims=True))
        cmin = jnp.min(d2, axis=0)
        bwd.append(jnp.mean(cmin * jax.lax.rsqrt(cmin + _TINY)))
    r = jnp.concatenate(rmins, axis=1)
    rt = r.T
    d_fwd = jnp.mean(rt * jax.lax.rsqrt(rt + _TINY), axis=1)
    d_bwd = jnp.stack(bwd)
    out_ref[...] = ((d_fwd + d_bwd) * 0.5).reshape(_BATCH_TILE, 1, 1)


def kernel(point_set_1, point_set_2):
    b, p1, _ = point_set_1.shape
    t = _BATCH_TILE
    coords = jnp.stack([point_set_1, point_set_2], axis=0)
    coords = coords.transpose(3, 0, 1, 2).reshape(4, b, p1)
    out = pl.pallas_call(
        _chamfer_body,
        grid=(b // t,),
        in_specs=[
            pl.BlockSpec((4, t, p1), lambda i: (0, i, 0)),
        ],
        out_specs=pl.BlockSpec((t, 1, 1), lambda i: (i, 0, 0)),
        out_shape=jax.ShapeDtypeStruct((b, 1, 1), jnp.float32),
    )(coords)
    return out.reshape(b)
